# Initial kernel scaffold; baseline (speedup 1.0000x reference)
import functools

import jax
import jax.numpy as jnp
from jax import lax
from jax.experimental import pallas as pl
from jax.experimental.pallas import tpu as pltpu
from jax.experimental.pallas import tpu_sc as plsc

NC = 2
NS = 16
NW = NC * NS
CHUNK = 80


def _sc_degree(dst3, n):
    nchunk = dst3.shape[1]
    mesh = plsc.VectorSubcoreMesh(core_axis_name="c", subcore_axis_name="s",
                                  num_cores=NC, num_subcores=NS)

    @functools.partial(
        pl.kernel,
        out_type=jax.ShapeDtypeStruct((NW, n), jnp.float32),
        mesh=mesh,
        scratch_types=[
            pltpu.VMEM((nchunk, CHUNK), jnp.int32),
            pltpu.VMEM((n,), jnp.float32),
        ],
    )
    def k(dst_hbm, out_hbm, dst_v, deg_v):
        c = lax.axis_index("c")
        s = lax.axis_index("s")
        wid = c * NS + s
        pltpu.sync_copy(dst_hbm.at[wid], dst_v)

        zeros16 = jnp.zeros((16,), jnp.float32)

        def zbody(i, carry):
            deg_v[pl.ds(i * 16, 16)] = zeros16
            return carry

        lax.fori_loop(0, n // 16, zbody, 0)

        ones16 = jnp.ones((16,), jnp.float32)

        def hbody(i, carry):
            for j in range(CHUNK // 16):
                idx = dst_v[i, pl.ds(j * 16, 16)]
                plsc.addupdate_scatter(deg_v, [idx], ones16)
            return carry

        lax.fori_loop(0, nchunk, hbody, 0)
        pltpu.sync_copy(deg_v, out_hbm.at[wid])

    return k(dst3)


def _sc_aggregate(vp, src3, dst3):
    n, F = vp.shape
    nchunk = src3.shape[1]
    rpt = n // NS
    mesh = plsc.VectorSubcoreMesh(core_axis_name="c", subcore_axis_name="s",
                                  num_cores=NC, num_subcores=NS)

    @functools.partial(
        pl.kernel,
        out_type=jax.ShapeDtypeStruct((NC, n, F), jnp.float32),
        mesh=mesh,
        scratch_types=[
            pltpu.VMEM((nchunk, CHUNK), jnp.int32),
            pltpu.VMEM((nchunk, CHUNK), jnp.int32),
            pltpu.VMEM((CHUNK, F), jnp.float32),
            pltpu.VMEM_SHARED((n, F), jnp.float32),
            pltpu.SemaphoreType.DMA,
        ],
    )
    def k(vp_hbm, src_hbm, dst_hbm, out_hbm, src_v, dst_v, rows_v, acc_sh, sem):
        c = lax.axis_index("c")
        s = lax.axis_index("s")
        wid = c * NS + s
        base = s * rpt
        pltpu.sync_copy(vp_hbm.at[pl.ds(base, rpt)], acc_sh.at[pl.ds(base, rpt)])
        pltpu.sync_copy(src_hbm.at[wid], src_v)
        pltpu.sync_copy(dst_hbm.at[wid], dst_v)
        plsc.subcore_barrier()

        def body(i, carry):
            pltpu.async_copy(vp_hbm.at[src_v.at[i]], rows_v, sem).wait()
            pltpu.sync_copy(rows_v, acc_sh.at[dst_v.at[i]], add=True)
            return carry

        lax.fori_loop(0, nchunk, body, 0)
        plsc.subcore_barrier()
        pltpu.sync_copy(acc_sh.at[pl.ds(base, rpt)],
                        out_hbm.at[c, pl.ds(base, rpt)])

    return k(vp, src3, dst3)


def _tc_prescale(degT, x):
    n, F = x.shape
    R = 1250

    def body(degT_ref, x_ref, dinv_ref, xp_ref):
        deg = jnp.sum(degT_ref[...], axis=1, keepdims=True) + 1.0
        dinv = lax.rsqrt(deg)
        dinv_ref[...] = dinv
        xp_ref[...] = x_ref[...] * dinv

    return pl.pallas_call(
        body,
        grid=(n // R,),
        in_specs=[pl.BlockSpec((R, NW), lambda i: (i, 0)),
                  pl.BlockSpec((R, F), lambda i: (i, 0))],
        out_specs=[pl.BlockSpec((R, 1), lambda i: (i, 0)),
                   pl.BlockSpec((R, F), lambda i: (i, 0))],
        out_shape=[jax.ShapeDtypeStruct((n, 1), jnp.float32),
                   jax.ShapeDtypeStruct((n, F), jnp.float32)],
    )(degT, x)


def _tc_mlp(a0, a1, xp, dinv, W1, b1, W2):
    n, F = xp.shape
    H = W1.shape[1]
    R = 1250

    def body(a0_ref, a1_ref, xp_ref, dinv_ref, W1_ref, b1_ref, W2_ref, gp_ref):
        agg = (a0_ref[...] + a1_ref[...] - xp_ref[...]) * dinv_ref[...]
        h = jnp.dot(agg, W1_ref[...], preferred_element_type=jnp.float32)
        h = jnp.maximum(h + b1_ref[...], 0.0)
        g = jnp.dot(h, W2_ref[...], preferred_element_type=jnp.float32)
        gp_ref[...] = g * dinv_ref[...]

    return pl.pallas_call(
        body,
        grid=(n // R,),
        in_specs=[pl.BlockSpec((R, F), lambda i: (i, 0)),
                  pl.BlockSpec((R, F), lambda i: (i, 0)),
                  pl.BlockSpec((R, F), lambda i: (i, 0)),
                  pl.BlockSpec((R, 1), lambda i: (i, 0)),
                  pl.BlockSpec((F, H), lambda i: (0, 0)),
                  pl.BlockSpec((1, H), lambda i: (0, 0)),
                  pl.BlockSpec((H, F), lambda i: (0, 0))],
        out_specs=pl.BlockSpec((R, F), lambda i: (i, 0)),
        out_shape=jax.ShapeDtypeStruct((n, F), jnp.float32),
    )(a0, a1, xp, dinv, W1, b1, W2)


def _tc_final(c0, c1, gp, dinv, b2):
    n, F = gp.shape
    R = 1250

    def body(c0_ref, c1_ref, gp_ref, dinv_ref, b2_ref, out_ref):
        agg = (c0_ref[...] + c1_ref[...] - gp_ref[...]) * dinv_ref[...]
        out_ref[...] = agg + b2_ref[...]

    return pl.pallas_call(
        body,
        grid=(n // R,),
        in_specs=[pl.BlockSpec((R, F), lambda i: (i, 0)),
                  pl.BlockSpec((R, F), lambda i: (i, 0)),
                  pl.BlockSpec((R, F), lambda i: (i, 0)),
                  pl.BlockSpec((R, 1), lambda i: (i, 0)),
                  pl.BlockSpec((1, F), lambda i: (0, 0))],
        out_specs=pl.BlockSpec((R, F), lambda i: (i, 0)),
        out_shape=jax.ShapeDtypeStruct((n, F), jnp.float32),
    )(c0, c1, gp, dinv, b2)


def kernel(x, edge_index, W1, b1, W2, b2):
    n, F = x.shape
    E = edge_index.shape[1]
    nchunk = E // (NW * CHUNK)
    src3 = edge_index[0].reshape(NW, nchunk, CHUNK)
    dst3 = edge_index[1].reshape(NW, nchunk, CHUNK)

    degp = _sc_degree(dst3, n)
    dinv, xp = _tc_prescale(degp.T, x)
    acc1 = _sc_aggregate(xp, src3, dst3)
    gp = _tc_mlp(acc1[0], acc1[1], xp, dinv,
                 W1, b1.reshape(1, -1), W2)
    acc2 = _sc_aggregate(gp, src3, dst3)
    out = _tc_final(acc2[0], acc2[1], gp, dinv, b2.reshape(1, -1))
    return out


# speedup vs baseline: 22.1989x; 22.1989x over previous
import functools

import jax
import jax.numpy as jnp
from jax import lax
from jax.experimental import pallas as pl
from jax.experimental.pallas import tpu as pltpu
from jax.experimental.pallas import tpu_sc as plsc

NC = 2
NS = 16
NW = NC * NS
CHUNK = 80


def _sc_degree(dst3, n):
    nchunk = dst3.shape[1]
    mesh = plsc.VectorSubcoreMesh(core_axis_name="c", subcore_axis_name="s",
                                  num_cores=NC, num_subcores=NS)

    @functools.partial(
        pl.kernel,
        out_type=jax.ShapeDtypeStruct((NW, n), jnp.float32),
        mesh=mesh,
        scratch_types=[
            pltpu.VMEM((nchunk, CHUNK), jnp.int32),
            pltpu.VMEM((n,), jnp.float32),
        ],
        compiler_params=pltpu.CompilerParams(needs_layout_passes=False),
    )
    def k(dst_hbm, out_hbm, dst_v, deg_v):
        c = lax.axis_index("c")
        s = lax.axis_index("s")
        wid = c * NS + s
        pltpu.sync_copy(dst_hbm.at[wid], dst_v)

        zeros16 = jnp.zeros((16,), jnp.float32)

        def zbody(i, carry):
            deg_v[pl.ds(i * 16, 16)] = zeros16
            return carry

        lax.fori_loop(0, n // 16, zbody, 0)

        ones16 = jnp.ones((16,), jnp.float32)

        def hbody(i, carry):
            for j in range(CHUNK // 16):
                idx = dst_v[i, pl.ds(j * 16, 16)]
                plsc.addupdate_scatter(deg_v, [idx], ones16)
            return carry

        lax.fori_loop(0, nchunk, hbody, 0)
        pltpu.sync_copy(deg_v, out_hbm.at[wid])

    return k(dst3)


def _sc_aggregate(vp, src3, dst3):
    n, F = vp.shape
    nchunk = src3.shape[1]
    NIO = 10
    rpt = n // NIO
    mesh = plsc.VectorSubcoreMesh(core_axis_name="c", subcore_axis_name="s",
                                  num_cores=NC, num_subcores=NS)

    @functools.partial(
        pl.kernel,
        out_type=jax.ShapeDtypeStruct((NC, n, F), jnp.float32),
        mesh=mesh,
        scratch_types=[
            pltpu.VMEM((nchunk, CHUNK), jnp.int32),
            pltpu.VMEM((nchunk, CHUNK), jnp.int32),
            pltpu.VMEM((CHUNK, F), jnp.float32),
            pltpu.VMEM_SHARED((n, F), jnp.float32),
            pltpu.SemaphoreType.DMA,
        ],
    )
    def k(vp_hbm, src_hbm, dst_hbm, out_hbm, src_v, dst_v, rows_v, acc_sh, sem):
        c = lax.axis_index("c")
        s = lax.axis_index("s")
        wid = c * NS + s
        base = s * rpt
        @pl.when(s < NIO)
        def _init():
            pltpu.sync_copy(vp_hbm.at[pl.ds(base, rpt)],
                            acc_sh.at[pl.ds(base, rpt)])
        pltpu.sync_copy(src_hbm.at[wid], src_v)
        pltpu.sync_copy(dst_hbm.at[wid], dst_v)
        plsc.subcore_barrier()

        def body(i, carry):
            pltpu.async_copy(vp_hbm.at[src_v.at[i]], rows_v, sem).wait()
            pltpu.sync_copy(rows_v, acc_sh.at[dst_v.at[i]], add=True)
            return carry

        lax.fori_loop(0, nchunk, body, 0)
        plsc.subcore_barrier()

        @pl.when(s < NIO)
        def _writeout():
            pltpu.sync_copy(acc_sh.at[pl.ds(base, rpt)],
                            out_hbm.at[c, pl.ds(base, rpt)])

    return k(vp, src3, dst3)


def _tc_prescale(degT, x):
    n, F = x.shape
    R = 1000

    def body(degT_ref, x_ref, dinv_ref, xp_ref):
        deg = jnp.sum(degT_ref[...], axis=1, keepdims=True) + 1.0
        dinv = lax.rsqrt(deg)
        dinv_ref[...] = dinv
        xp_ref[...] = x_ref[...] * dinv

    return pl.pallas_call(
        body,
        grid=(n // R,),
        in_specs=[pl.BlockSpec((R, NW), lambda i: (i, 0)),
                  pl.BlockSpec((R, F), lambda i: (i, 0))],
        out_specs=[pl.BlockSpec((R, 1), lambda i: (i, 0)),
                   pl.BlockSpec((R, F), lambda i: (i, 0))],
        out_shape=[jax.ShapeDtypeStruct((n, 1), jnp.float32),
                   jax.ShapeDtypeStruct((n, F), jnp.float32)],
    )(degT, x)


def _tc_mlp(a0, a1, xp, dinv, W1, b1, W2):
    n, F = xp.shape
    H = W1.shape[1]
    R = 1000

    def body(a0_ref, a1_ref, xp_ref, dinv_ref, W1_ref, b1_ref, W2_ref, gp_ref):
        agg = (a0_ref[...] + a1_ref[...] - xp_ref[...]) * dinv_ref[...]
        h = jnp.dot(agg, W1_ref[...], preferred_element_type=jnp.float32)
        h = jnp.maximum(h + b1_ref[...], 0.0)
        g = jnp.dot(h, W2_ref[...], preferred_element_type=jnp.float32)
        gp_ref[...] = g * dinv_ref[...]

    return pl.pallas_call(
        body,
        grid=(n // R,),
        in_specs=[pl.BlockSpec((R, F), lambda i: (i, 0)),
                  pl.BlockSpec((R, F), lambda i: (i, 0)),
                  pl.BlockSpec((R, F), lambda i: (i, 0)),
                  pl.BlockSpec((R, 1), lambda i: (i, 0)),
                  pl.BlockSpec((F, H), lambda i: (0, 0)),
                  pl.BlockSpec((1, H), lambda i: (0, 0)),
                  pl.BlockSpec((H, F), lambda i: (0, 0))],
        out_specs=pl.BlockSpec((R, F), lambda i: (i, 0)),
        out_shape=jax.ShapeDtypeStruct((n, F), jnp.float32),
    )(a0, a1, xp, dinv, W1, b1, W2)


def _tc_final(c0, c1, gp, dinv, b2):
    n, F = gp.shape
    R = 1000

    def body(c0_ref, c1_ref, gp_ref, dinv_ref, b2_ref, out_ref):
        agg = (c0_ref[...] + c1_ref[...] - gp_ref[...]) * dinv_ref[...]
        out_ref[...] = agg + b2_ref[...]

    return pl.pallas_call(
        body,
        grid=(n // R,),
        in_specs=[pl.BlockSpec((R, F), lambda i: (i, 0)),
                  pl.BlockSpec((R, F), lambda i: (i, 0)),
                  pl.BlockSpec((R, F), lambda i: (i, 0)),
                  pl.BlockSpec((R, 1), lambda i: (i, 0)),
                  pl.BlockSpec((1, F), lambda i: (0, 0))],
        out_specs=pl.BlockSpec((R, F), lambda i: (i, 0)),
        out_shape=jax.ShapeDtypeStruct((n, F), jnp.float32),
    )(c0, c1, gp, dinv, b2)


def kernel(x, edge_index, W1, b1, W2, b2):
    n, F = x.shape
    E = edge_index.shape[1]
    nchunk = E // (NW * CHUNK)
    src3 = edge_index[0].reshape(NW, nchunk, CHUNK)
    dst3 = edge_index[1].reshape(NW, nchunk, CHUNK)

    degp = _sc_degree(dst3, n)
    dinv, xp = _tc_prescale(degp.T, x)
    acc1 = _sc_aggregate(xp, src3, dst3)
    gp = _tc_mlp(acc1[0], acc1[1], xp, dinv,
                 W1, b1.reshape(1, -1), W2)
    acc2 = _sc_aggregate(gp, src3, dst3)
    out = _tc_final(acc2[0], acc2[1], gp, dinv, b2.reshape(1, -1))
    return out


# speedup vs baseline: 26.8318x; 1.2087x over previous
import functools

import jax
import jax.numpy as jnp
from jax import lax
from jax.experimental import pallas as pl
from jax.experimental.pallas import tpu as pltpu
from jax.experimental.pallas import tpu_sc as plsc

NC = 2
NS = 16
NW = NC * NS
CHUNK = 80
SB = 5
SBC = 25


def _sc_degree(dst3, n):
    nchunk = dst3.shape[1]
    mesh = plsc.VectorSubcoreMesh(core_axis_name="c", subcore_axis_name="s",
                                  num_cores=NC, num_subcores=NS)

    @functools.partial(
        pl.kernel,
        out_type=jax.ShapeDtypeStruct((NW, n), jnp.float32),
        mesh=mesh,
        scratch_types=[
            pltpu.VMEM((nchunk, CHUNK), jnp.int32),
            pltpu.VMEM((n,), jnp.float32),
        ],
        compiler_params=pltpu.CompilerParams(needs_layout_passes=False),
    )
    def k(dst_hbm, out_hbm, dst_v, deg_v):
        c = lax.axis_index("c")
        s = lax.axis_index("s")
        wid = c * NS + s
        pltpu.sync_copy(dst_hbm.at[wid], dst_v)

        zeros16 = jnp.zeros((16,), jnp.float32)

        def zbody(i, carry):
            deg_v[pl.ds(i * 16, 16)] = zeros16
            return carry

        lax.fori_loop(0, n // 16, zbody, 0)

        ones16 = jnp.ones((16,), jnp.float32)

        def hbody(i, carry):
            for j in range(CHUNK // 16):
                idx = dst_v[i, pl.ds(j * 16, 16)]
                plsc.addupdate_scatter(deg_v, [idx], ones16)
            return carry

        lax.fori_loop(0, nchunk, hbody, 0)
        pltpu.sync_copy(deg_v, out_hbm.at[wid])

    return k(dst3)


def _sc_aggregate(vp, src4, dst4):
    n, F = vp.shape
    NIO = 10
    rpt = n // NIO
    mesh = plsc.VectorSubcoreMesh(core_axis_name="c", subcore_axis_name="s",
                                  num_cores=NC, num_subcores=NS)

    @functools.partial(
        pl.kernel,
        out_type=jax.ShapeDtypeStruct((NC, n, F), jnp.float32),
        mesh=mesh,
        scratch_types=[
            pltpu.VMEM((SBC, CHUNK), jnp.int32),
            pltpu.VMEM((SBC, CHUNK), jnp.int32),
            pltpu.VMEM((CHUNK, F), jnp.float32),
            pltpu.VMEM((CHUNK, F), jnp.float32),
            pltpu.VMEM_SHARED((n, F), jnp.float32),
            pltpu.SemaphoreType.DMA,
            pltpu.SemaphoreType.DMA,
        ],
    )
    def k(vp_hbm, src_hbm, dst_hbm, out_hbm, src_v, dst_v, rows0, rows1,
          acc_sh, sem0, sem1):
        c = lax.axis_index("c")
        s = lax.axis_index("s")
        wid = c * NS + s
        base = s * rpt
        @pl.when(s < NIO)
        def _init():
            pltpu.sync_copy(vp_hbm.at[pl.ds(base, rpt)],
                            acc_sh.at[pl.ds(base, rpt)])
        plsc.subcore_barrier()

        def sblock(sb, carry):
            pltpu.sync_copy(src_hbm.at[wid, sb], src_v)
            pltpu.sync_copy(dst_hbm.at[wid, sb], dst_v)
            pltpu.async_copy(vp_hbm.at[src_v.at[0]], rows0, sem0)

            def body(k, carry2):
                i0 = 2 * k
                pltpu.make_async_copy(vp_hbm.at[src_v.at[i0]], rows0, sem0).wait()
                pltpu.async_copy(vp_hbm.at[src_v.at[i0 + 1]], rows1, sem1)
                pltpu.sync_copy(rows0, acc_sh.at[dst_v.at[i0]], add=True)
                pltpu.make_async_copy(vp_hbm.at[src_v.at[i0 + 1]], rows1,
                                      sem1).wait()
                pltpu.async_copy(vp_hbm.at[src_v.at[i0 + 2]], rows0, sem0)
                pltpu.sync_copy(rows1, acc_sh.at[dst_v.at[i0 + 1]], add=True)
                return carry2

            lax.fori_loop(0, (SBC - 1) // 2, body, 0)
            pltpu.make_async_copy(vp_hbm.at[src_v.at[SBC - 1]], rows0,
                                  sem0).wait()
            pltpu.sync_copy(rows0, acc_sh.at[dst_v.at[SBC - 1]], add=True)
            return carry

        lax.fori_loop(0, SB, sblock, 0)
        plsc.subcore_barrier()

        @pl.when(s < NIO)
        def _writeout():
            pltpu.sync_copy(acc_sh.at[pl.ds(base, rpt)],
                            out_hbm.at[c, pl.ds(base, rpt)])

    return k(vp, src4, dst4)


def _tc_prescale(degT, x):
    n, F = x.shape
    R = 1000

    def body(degT_ref, x_ref, dinv_ref, xp_ref):
        deg = jnp.sum(degT_ref[...], axis=1, keepdims=True) + 1.0
        dinv = lax.rsqrt(deg)
        dinv_ref[...] = dinv
        xp_ref[...] = x_ref[...] * dinv

    return pl.pallas_call(
        body,
        grid=(n // R,),
        in_specs=[pl.BlockSpec((R, NW), lambda i: (i, 0)),
                  pl.BlockSpec((R, F), lambda i: (i, 0))],
        out_specs=[pl.BlockSpec((R, 1), lambda i: (i, 0)),
                   pl.BlockSpec((R, F), lambda i: (i, 0))],
        out_shape=[jax.ShapeDtypeStruct((n, 1), jnp.float32),
                   jax.ShapeDtypeStruct((n, F), jnp.float32)],
    )(degT, x)


def _tc_mlp(a0, a1, xp, dinv, W1, b1, W2):
    n, F = xp.shape
    H = W1.shape[1]
    R = 1000

    def body(a0_ref, a1_ref, xp_ref, dinv_ref, W1_ref, b1_ref, W2_ref, gp_ref):
        agg = (a0_ref[...] + a1_ref[...] - xp_ref[...]) * dinv_ref[...]
        h = jnp.dot(agg, W1_ref[...], preferred_element_type=jnp.float32)
        h = jnp.maximum(h + b1_ref[...], 0.0)
        g = jnp.dot(h, W2_ref[...], preferred_element_type=jnp.float32)
        gp_ref[...] = g * dinv_ref[...]

    return pl.pallas_call(
        body,
        grid=(n // R,),
        in_specs=[pl.BlockSpec((R, F), lambda i: (i, 0)),
                  pl.BlockSpec((R, F), lambda i: (i, 0)),
                  pl.BlockSpec((R, F), lambda i: (i, 0)),
                  pl.BlockSpec((R, 1), lambda i: (i, 0)),
                  pl.BlockSpec((F, H), lambda i: (0, 0)),
                  pl.BlockSpec((1, H), lambda i: (0, 0)),
                  pl.BlockSpec((H, F), lambda i: (0, 0))],
        out_specs=pl.BlockSpec((R, F), lambda i: (i, 0)),
        out_shape=jax.ShapeDtypeStruct((n, F), jnp.float32),
    )(a0, a1, xp, dinv, W1, b1, W2)


def _tc_final(c0, c1, gp, dinv, b2):
    n, F = gp.shape
    R = 1000

    def body(c0_ref, c1_ref, gp_ref, dinv_ref, b2_ref, out_ref):
        agg = (c0_ref[...] + c1_ref[...] - gp_ref[...]) * dinv_ref[...]
        out_ref[...] = agg + b2_ref[...]

    return pl.pallas_call(
        body,
        grid=(n // R,),
        in_specs=[pl.BlockSpec((R, F), lambda i: (i, 0)),
                  pl.BlockSpec((R, F), lambda i: (i, 0)),
                  pl.BlockSpec((R, F), lambda i: (i, 0)),
                  pl.BlockSpec((R, 1), lambda i: (i, 0)),
                  pl.BlockSpec((1, F), lambda i: (0, 0))],
        out_specs=pl.BlockSpec((R, F), lambda i: (i, 0)),
        out_shape=jax.ShapeDtypeStruct((n, F), jnp.float32),
    )(c0, c1, gp, dinv, b2)


def kernel(x, edge_index, W1, b1, W2, b2):
    n, F = x.shape
    src4 = edge_index[0].reshape(NW, SB, SBC, CHUNK)
    dst4 = edge_index[1].reshape(NW, SB, SBC, CHUNK)
    dst3 = edge_index[1].reshape(NW, SB * SBC, CHUNK)

    degp = _sc_degree(dst3, n)
    dinv, xp = _tc_prescale(degp.T, x)
    acc1 = _sc_aggregate(xp, src4, dst4)
    gp = _tc_mlp(acc1[0], acc1[1], xp, dinv,
                 W1, b1.reshape(1, -1), W2)
    acc2 = _sc_aggregate(gp, src4, dst4)
    out = _tc_final(acc2[0], acc2[1], gp, dinv, b2.reshape(1, -1))
    return out


# speedup vs baseline: 29.5047x; 1.0996x over previous
import functools

import jax
import jax.numpy as jnp
from jax import lax
from jax.experimental import pallas as pl
from jax.experimental.pallas import tpu as pltpu
from jax.experimental.pallas import tpu_sc as plsc

NC = 2
NS = 16
NW = NC * NS
DCHUNK = 80
CHUNK = 128
SB = 6
SBC = 13
NEX = 4


def _sc_degree(dst3, n):
    nchunk = dst3.shape[1]
    mesh = plsc.VectorSubcoreMesh(core_axis_name="c", subcore_axis_name="s",
                                  num_cores=NC, num_subcores=NS)

    @functools.partial(
        pl.kernel,
        out_type=jax.ShapeDtypeStruct((NW, n), jnp.float32),
        mesh=mesh,
        scratch_types=[
            pltpu.VMEM((nchunk, DCHUNK), jnp.int32),
            pltpu.VMEM((n,), jnp.float32),
        ],
        compiler_params=pltpu.CompilerParams(needs_layout_passes=False),
    )
    def k(dst_hbm, out_hbm, dst_v, deg_v):
        c = lax.axis_index("c")
        s = lax.axis_index("s")
        wid = c * NS + s
        pltpu.sync_copy(dst_hbm.at[wid], dst_v)

        zeros16 = jnp.zeros((16,), jnp.float32)

        def zbody(i, carry):
            deg_v[pl.ds(i * 16, 16)] = zeros16
            return carry

        lax.fori_loop(0, n // 16, zbody, 0)

        ones16 = jnp.ones((16,), jnp.float32)

        def hbody(i, carry):
            for j in range(DCHUNK // 16):
                idx = dst_v[i, pl.ds(j * 16, 16)]
                plsc.addupdate_scatter(deg_v, [idx], ones16)
            return carry

        lax.fori_loop(0, nchunk, hbody, 0)
        pltpu.sync_copy(deg_v, out_hbm.at[wid])

    return k(dst3)


def _sc_aggregate(vp, src4, dst4, esrc, edst):
    n, F = vp.shape
    NIO = 10
    rpt = n // NIO
    mesh = plsc.VectorSubcoreMesh(core_axis_name="c", subcore_axis_name="s",
                                  num_cores=NC, num_subcores=NS)

    @functools.partial(
        pl.kernel,
        out_type=jax.ShapeDtypeStruct((NC, n, F), jnp.float32),
        mesh=mesh,
        scratch_types=[
            pltpu.VMEM((SBC, CHUNK), jnp.int32),
            pltpu.VMEM((SBC, CHUNK), jnp.int32),
            pltpu.VMEM((1, CHUNK), jnp.int32),
            pltpu.VMEM((1, CHUNK), jnp.int32),
            pltpu.VMEM((CHUNK, F), jnp.float32),
            pltpu.VMEM((CHUNK, F), jnp.float32),
            pltpu.VMEM_SHARED((n, F), jnp.float32),
            pltpu.SemaphoreType.DMA,
            pltpu.SemaphoreType.DMA,
        ],
    )
    def k(vp_hbm, src_hbm, dst_hbm, esrc_hbm, edst_hbm, out_hbm,
          src_v, dst_v, es_v, ed_v, rows0, rows1, acc_sh, sem0, sem1):
        c = lax.axis_index("c")
        s = lax.axis_index("s")
        wid = c * NS + s
        base = s * rpt
        @pl.when(s < NIO)
        def _init():
            pltpu.sync_copy(vp_hbm.at[pl.ds(base, rpt)],
                            acc_sh.at[pl.ds(base, rpt)])
        plsc.subcore_barrier()

        def sblock(sb, carry):
            pltpu.sync_copy(src_hbm.at[wid, sb], src_v)
            pltpu.sync_copy(dst_hbm.at[wid, sb], dst_v)
            pltpu.async_copy(vp_hbm.at[src_v.at[0]], rows0, sem0)

            def body(k, carry2):
                i0 = 2 * k
                pltpu.make_async_copy(vp_hbm.at[src_v.at[i0]], rows0, sem0).wait()
                pltpu.async_copy(vp_hbm.at[src_v.at[i0 + 1]], rows1, sem1)
                pltpu.sync_copy(rows0, acc_sh.at[dst_v.at[i0]], add=True)
                pltpu.make_async_copy(vp_hbm.at[src_v.at[i0 + 1]], rows1,
                                      sem1).wait()
                pltpu.async_copy(vp_hbm.at[src_v.at[i0 + 2]], rows0, sem0)
                pltpu.sync_copy(rows1, acc_sh.at[dst_v.at[i0 + 1]], add=True)
                return carry2

            lax.fori_loop(0, (SBC - 1) // 2, body, 0)
            pltpu.make_async_copy(vp_hbm.at[src_v.at[SBC - 1]], rows0,
                                  sem0).wait()
            pltpu.sync_copy(rows0, acc_sh.at[dst_v.at[SBC - 1]], add=True)
            return carry

        lax.fori_loop(0, SB, sblock, 0)

        @pl.when(s < NEX // NC)
        def _extra():
            e = c * (NEX // NC) + s
            pltpu.sync_copy(esrc_hbm.at[e], es_v.at[0])
            pltpu.sync_copy(edst_hbm.at[e], ed_v.at[0])
            pltpu.async_copy(vp_hbm.at[es_v.at[0]], rows0, sem0).wait()
            pltpu.sync_copy(rows0, acc_sh.at[ed_v.at[0]], add=True)

        plsc.subcore_barrier()

        @pl.when(s < NIO)
        def _writeout():
            pltpu.sync_copy(acc_sh.at[pl.ds(base, rpt)],
                            out_hbm.at[c, pl.ds(base, rpt)])

    return k(vp, src4, dst4, esrc, edst)


def _tc_prescale(degT, x):
    n, F = x.shape
    R = 1000

    def body(degT_ref, x_ref, dinv_ref, xp_ref):
        deg = jnp.sum(degT_ref[...], axis=1, keepdims=True) + 1.0
        dinv = lax.rsqrt(deg)
        dinv_ref[...] = dinv
        xp_ref[...] = x_ref[...] * dinv

    return pl.pallas_call(
        body,
        grid=(n // R,),
        in_specs=[pl.BlockSpec((R, NW), lambda i: (i, 0)),
                  pl.BlockSpec((R, F), lambda i: (i, 0))],
        out_specs=[pl.BlockSpec((R, 1), lambda i: (i, 0)),
                   pl.BlockSpec((R, F), lambda i: (i, 0))],
        out_shape=[jax.ShapeDtypeStruct((n, 1), jnp.float32),
                   jax.ShapeDtypeStruct((n, F), jnp.float32)],
    )(degT, x)


def _tc_mlp(a0, a1, xp, dinv, W1, b1, W2):
    n, F = xp.shape
    H = W1.shape[1]
    R = 1000

    def body(a0_ref, a1_ref, xp_ref, dinv_ref, W1_ref, b1_ref, W2_ref, gp_ref):
        agg = (a0_ref[...] + a1_ref[...] - xp_ref[...]) * dinv_ref[...]
        h = jnp.dot(agg, W1_ref[...], preferred_element_type=jnp.float32)
        h = jnp.maximum(h + b1_ref[...], 0.0)
        g = jnp.dot(h, W2_ref[...], preferred_element_type=jnp.float32)
        gp_ref[...] = g * dinv_ref[...]

    return pl.pallas_call(
        body,
        grid=(n // R,),
        in_specs=[pl.BlockSpec((R, F), lambda i: (i, 0)),
                  pl.BlockSpec((R, F), lambda i: (i, 0)),
                  pl.BlockSpec((R, F), lambda i: (i, 0)),
                  pl.BlockSpec((R, 1), lambda i: (i, 0)),
                  pl.BlockSpec((F, H), lambda i: (0, 0)),
                  pl.BlockSpec((1, H), lambda i: (0, 0)),
                  pl.BlockSpec((H, F), lambda i: (0, 0))],
        out_specs=pl.BlockSpec((R, F), lambda i: (i, 0)),
        out_shape=jax.ShapeDtypeStruct((n, F), jnp.float32),
    )(a0, a1, xp, dinv, W1, b1, W2)


def _tc_final(c0, c1, gp, dinv, b2):
    n, F = gp.shape
    R = 1000

    def body(c0_ref, c1_ref, gp_ref, dinv_ref, b2_ref, out_ref):
        agg = (c0_ref[...] + c1_ref[...] - gp_ref[...]) * dinv_ref[...]
        out_ref[...] = agg + b2_ref[...]

    return pl.pallas_call(
        body,
        grid=(n // R,),
        in_specs=[pl.BlockSpec((R, F), lambda i: (i, 0)),
                  pl.BlockSpec((R, F), lambda i: (i, 0)),
                  pl.BlockSpec((R, F), lambda i: (i, 0)),
                  pl.BlockSpec((R, 1), lambda i: (i, 0)),
                  pl.BlockSpec((1, F), lambda i: (0, 0))],
        out_specs=pl.BlockSpec((R, F), lambda i: (i, 0)),
        out_shape=jax.ShapeDtypeStruct((n, F), jnp.float32),
    )(c0, c1, gp, dinv, b2)


def kernel(x, edge_index, W1, b1, W2, b2):
    n, F = x.shape
    E = edge_index.shape[1]
    emain = NW * SB * SBC * CHUNK
    src4 = edge_index[0, :emain].reshape(NW, SB, SBC, CHUNK)
    dst4 = edge_index[1, :emain].reshape(NW, SB, SBC, CHUNK)
    esrc = edge_index[0, emain:].reshape(NEX, CHUNK)
    edst = edge_index[1, emain:].reshape(NEX, CHUNK)
    dst3 = edge_index[1].reshape(NW, E // (NW * DCHUNK), DCHUNK)

    degp = _sc_degree(dst3, n)
    dinv, xp = _tc_prescale(degp.T, x)
    acc1 = _sc_aggregate(xp, src4, dst4, esrc, edst)
    gp = _tc_mlp(acc1[0], acc1[1], xp, dinv,
                 W1, b1.reshape(1, -1), W2)
    acc2 = _sc_aggregate(gp, src4, dst4, esrc, edst)
    out = _tc_final(acc2[0], acc2[1], gp, dinv, b2.reshape(1, -1))
    return out


# speedup vs baseline: 30.9397x; 1.0486x over previous
import functools

import jax
import jax.numpy as jnp
from jax import lax
from jax.experimental import pallas as pl
from jax.experimental.pallas import tpu as pltpu
from jax.experimental.pallas import tpu_sc as plsc

NC = 2
NS = 16
NW = NC * NS
DCHUNK = 80
CHUNK = 128
SB = 6
SBC = 13
NEX = 4


def _sc_degree(dst4, edst, n):
    mesh = plsc.VectorSubcoreMesh(core_axis_name="c", subcore_axis_name="s",
                                  num_cores=NC, num_subcores=NS)

    @functools.partial(
        pl.kernel,
        out_type=jax.ShapeDtypeStruct((NW, n), jnp.float32),
        mesh=mesh,
        scratch_types=[
            pltpu.VMEM((SBC, CHUNK), jnp.int32),
            pltpu.VMEM((1, CHUNK), jnp.int32),
            pltpu.VMEM((n,), jnp.float32),
        ],
        compiler_params=pltpu.CompilerParams(needs_layout_passes=False),
    )
    def k(dst_hbm, edst_hbm, out_hbm, dst_v, ed_v, deg_v):
        c = lax.axis_index("c")
        s = lax.axis_index("s")
        wid = c * NS + s

        zeros16 = jnp.zeros((16,), jnp.float32)

        def zbody(i, carry):
            deg_v[pl.ds(i * 16, 16)] = zeros16
            return carry

        lax.fori_loop(0, n // 16, zbody, 0)

        ones16 = jnp.ones((16,), jnp.float32)

        def sblock(sb, carry):
            pltpu.sync_copy(dst_hbm.at[wid, sb], dst_v)

            def hbody(i, carry2):
                for j in range(CHUNK // 16):
                    idx = dst_v[i, pl.ds(j * 16, 16)]
                    plsc.addupdate_scatter(deg_v, [idx], ones16)
                return carry2

            lax.fori_loop(0, SBC, hbody, 0)
            return carry

        lax.fori_loop(0, SB, sblock, 0)

        @pl.when(s < NEX // NC)
        def _extra():
            pltpu.sync_copy(edst_hbm.at[c * (NEX // NC) + s], ed_v.at[0])

            def ebody(j, carry2):
                idx = ed_v[0, pl.ds(j * 16, 16)]
                plsc.addupdate_scatter(deg_v, [idx], ones16)
                return carry2

            lax.fori_loop(0, CHUNK // 16, ebody, 0)

        pltpu.sync_copy(deg_v, out_hbm.at[wid])

    return k(dst4, edst)


def _sc_aggregate(vp, zeros, src4, dst4, esrc, edst):
    n, F = vp.shape
    NIO = 10
    rpt = n // NIO
    mesh = plsc.VectorSubcoreMesh(core_axis_name="c", subcore_axis_name="s",
                                  num_cores=NC, num_subcores=NS)

    @functools.partial(
        pl.kernel,
        out_type=jax.ShapeDtypeStruct((NC, n, F), jnp.float32),
        mesh=mesh,
        scratch_types=[
            pltpu.VMEM((SBC, CHUNK), jnp.int32),
            pltpu.VMEM((SBC, CHUNK), jnp.int32),
            pltpu.VMEM((1, CHUNK), jnp.int32),
            pltpu.VMEM((1, CHUNK), jnp.int32),
            pltpu.VMEM((CHUNK, F), jnp.float32),
            pltpu.VMEM((CHUNK, F), jnp.float32),
            pltpu.VMEM_SHARED((n, F), jnp.float32),
            pltpu.SemaphoreType.DMA,
            pltpu.SemaphoreType.DMA,
        ],
    )
    def k(vp_hbm, z_hbm, src_hbm, dst_hbm, esrc_hbm, edst_hbm, out_hbm,
          src_v, dst_v, es_v, ed_v, rows0, rows1, acc_sh, sem0, sem1):
        c = lax.axis_index("c")
        s = lax.axis_index("s")
        wid = c * NS + s
        base = s * rpt
        @pl.when((c == 0) & (s < NIO))
        def _init0():
            pltpu.sync_copy(vp_hbm.at[pl.ds(base, rpt)],
                            acc_sh.at[pl.ds(base, rpt)])

        @pl.when((c == 1) & (s < NIO))
        def _init1():
            pltpu.sync_copy(z_hbm.at[pl.ds(base, rpt)],
                            acc_sh.at[pl.ds(base, rpt)])
        plsc.subcore_barrier()

        def sblock(sb, carry):
            pltpu.sync_copy(src_hbm.at[wid, sb], src_v)
            pltpu.sync_copy(dst_hbm.at[wid, sb], dst_v)
            pltpu.async_copy(vp_hbm.at[src_v.at[0]], rows0, sem0)

            def body(k, carry2):
                i0 = 2 * k
                pltpu.make_async_copy(vp_hbm.at[src_v.at[i0]], rows0, sem0).wait()
                pltpu.async_copy(vp_hbm.at[src_v.at[i0 + 1]], rows1, sem1)
                pltpu.sync_copy(rows0, acc_sh.at[dst_v.at[i0]], add=True)
                pltpu.make_async_copy(vp_hbm.at[src_v.at[i0 + 1]], rows1,
                                      sem1).wait()
                pltpu.async_copy(vp_hbm.at[src_v.at[i0 + 2]], rows0, sem0)
                pltpu.sync_copy(rows1, acc_sh.at[dst_v.at[i0 + 1]], add=True)
                return carry2

            lax.fori_loop(0, (SBC - 1) // 2, body, 0)
            pltpu.make_async_copy(vp_hbm.at[src_v.at[SBC - 1]], rows0,
                                  sem0).wait()
            pltpu.sync_copy(rows0, acc_sh.at[dst_v.at[SBC - 1]], add=True)
            return carry

        lax.fori_loop(0, SB, sblock, 0)

        @pl.when(s < NEX // NC)
        def _extra():
            e = c * (NEX // NC) + s
            pltpu.sync_copy(esrc_hbm.at[e], es_v.at[0])
            pltpu.sync_copy(edst_hbm.at[e], ed_v.at[0])
            pltpu.async_copy(vp_hbm.at[es_v.at[0]], rows0, sem0).wait()
            pltpu.sync_copy(rows0, acc_sh.at[ed_v.at[0]], add=True)

        plsc.subcore_barrier()

        @pl.when(s < NIO)
        def _writeout():
            pltpu.sync_copy(acc_sh.at[pl.ds(base, rpt)],
                            out_hbm.at[c, pl.ds(base, rpt)])

    return k(vp, zeros, src4, dst4, esrc, edst)


def _tc_prescale(degT, x):
    n, F = x.shape
    R = 1000

    def body(degT_ref, x_ref, dinv_ref, xp_ref):
        deg = jnp.sum(degT_ref[...], axis=1, keepdims=True) + 1.0
        dinv = lax.rsqrt(deg)
        dinv_ref[...] = dinv
        xp_ref[...] = x_ref[...] * dinv

    return pl.pallas_call(
        body,
        grid=(n // R,),
        in_specs=[pl.BlockSpec((R, NW), lambda i: (i, 0)),
                  pl.BlockSpec((R, F), lambda i: (i, 0))],
        out_specs=[pl.BlockSpec((R, 1), lambda i: (i, 0)),
                   pl.BlockSpec((R, F), lambda i: (i, 0))],
        out_shape=[jax.ShapeDtypeStruct((n, 1), jnp.float32),
                   jax.ShapeDtypeStruct((n, F), jnp.float32)],
    )(degT, x)


def _tc_mlp(acc, dinv, W1, b1, W2):
    _, n, F = acc.shape
    H = W1.shape[1]
    R = 1000

    def body(a0_ref, a1_ref, dinv_ref, W1_ref, b1_ref, W2_ref, gp_ref):
        agg = (a0_ref[0] + a1_ref[0]) * dinv_ref[...]
        h = jnp.dot(agg, W1_ref[...], preferred_element_type=jnp.float32)
        h = jnp.maximum(h + b1_ref[...], 0.0)
        g = jnp.dot(h, W2_ref[...], preferred_element_type=jnp.float32)
        gp_ref[...] = g * dinv_ref[...]

    return pl.pallas_call(
        body,
        grid=(n // R,),
        in_specs=[pl.BlockSpec((1, R, F), lambda i: (0, i, 0)),
                  pl.BlockSpec((1, R, F), lambda i: (1, i, 0)),
                  pl.BlockSpec((R, 1), lambda i: (i, 0)),
                  pl.BlockSpec((F, H), lambda i: (0, 0)),
                  pl.BlockSpec((1, H), lambda i: (0, 0)),
                  pl.BlockSpec((H, F), lambda i: (0, 0))],
        out_specs=pl.BlockSpec((R, F), lambda i: (i, 0)),
        out_shape=jax.ShapeDtypeStruct((n, F), jnp.float32),
    )(acc, acc, dinv, W1, b1, W2)


def _tc_final(acc, dinv, b2):
    _, n, F = acc.shape
    R = 1000

    def body(c0_ref, c1_ref, dinv_ref, b2_ref, out_ref):
        agg = (c0_ref[0] + c1_ref[0]) * dinv_ref[...]
        out_ref[...] = agg + b2_ref[...]

    return pl.pallas_call(
        body,
        grid=(n // R,),
        in_specs=[pl.BlockSpec((1, R, F), lambda i: (0, i, 0)),
                  pl.BlockSpec((1, R, F), lambda i: (1, i, 0)),
                  pl.BlockSpec((R, 1), lambda i: (i, 0)),
                  pl.BlockSpec((1, F), lambda i: (0, 0))],
        out_specs=pl.BlockSpec((R, F), lambda i: (i, 0)),
        out_shape=jax.ShapeDtypeStruct((n, F), jnp.float32),
    )(acc, acc, dinv, b2)


def kernel(x, edge_index, W1, b1, W2, b2):
    n, F = x.shape
    emain = NW * SB * SBC * CHUNK
    src4 = edge_index[0, :emain].reshape(NW, SB, SBC, CHUNK)
    dst4 = edge_index[1, :emain].reshape(NW, SB, SBC, CHUNK)
    esrc = edge_index[0, emain:].reshape(NEX, CHUNK)
    edst = edge_index[1, emain:].reshape(NEX, CHUNK)
    zeros = jnp.zeros((n, F), jnp.float32)

    degp = _sc_degree(dst4, edst, n)
    dinv, xp = _tc_prescale(degp.T, x)
    acc1 = _sc_aggregate(xp, zeros, src4, dst4, esrc, edst)
    gp = _tc_mlp(acc1, dinv, W1, b1.reshape(1, -1), W2)
    acc2 = _sc_aggregate(gp, zeros, src4, dst4, esrc, edst)
    out = _tc_final(acc2, dinv, b2.reshape(1, -1))
    return out


# speedup vs baseline: 36.8486x; 1.1910x over previous
import functools

import jax
import jax.numpy as jnp
from jax import lax
from jax.experimental import pallas as pl
from jax.experimental.pallas import tpu as pltpu
from jax.experimental.pallas import tpu_sc as plsc

NC = 2
NS = 16
NW = NC * NS
CHUNK = 80
SB = 5
SBC = 25


def _sc_degree(dst4, n):
    mesh = plsc.VectorSubcoreMesh(core_axis_name="c", subcore_axis_name="s",
                                  num_cores=NC, num_subcores=NS)

    @functools.partial(
        pl.kernel,
        out_type=jax.ShapeDtypeStruct((NW, n), jnp.float32),
        mesh=mesh,
        scratch_types=[
            pltpu.VMEM((SBC, CHUNK), jnp.int32),
            pltpu.VMEM((n,), jnp.float32),
        ],
        compiler_params=pltpu.CompilerParams(needs_layout_passes=False),
    )
    def k(dst_hbm, out_hbm, dst_v, deg_v):
        c = lax.axis_index("c")
        s = lax.axis_index("s")
        wid = c * NS + s

        zeros16 = jnp.zeros((16,), jnp.float32)

        def zbody(i, carry):
            deg_v[pl.ds(i * 16, 16)] = zeros16
            return carry

        lax.fori_loop(0, n // 16, zbody, 0)

        ones16 = jnp.ones((16,), jnp.float32)

        def sblock(sb, carry):
            pltpu.sync_copy(dst_hbm.at[wid, sb], dst_v)

            def hbody(i, carry2):
                for j in range(CHUNK // 16):
                    idx = dst_v[i, pl.ds(j * 16, 16)]
                    plsc.addupdate_scatter(deg_v, [idx], ones16)
                return carry2

            lax.fori_loop(0, SBC, hbody, 0)
            return carry

        lax.fori_loop(0, SB, sblock, 0)
        pltpu.sync_copy(deg_v, out_hbm.at[wid])

    return k(dst4)


def _sc_aggregate(vp, zeros, src4, dst4):
    n, F = vp.shape
    NIO = 10
    rpt = n // NIO
    mesh = plsc.VectorSubcoreMesh(core_axis_name="c", subcore_axis_name="s",
                                  num_cores=NC, num_subcores=NS)

    @functools.partial(
        pl.kernel,
        out_type=jax.ShapeDtypeStruct((NC, n, F), jnp.float32),
        mesh=mesh,
        scratch_types=[
            pltpu.VMEM((SBC, CHUNK), jnp.int32),
            pltpu.VMEM((SBC, CHUNK), jnp.int32),
            pltpu.VMEM((CHUNK, F), jnp.float32),
            pltpu.VMEM((CHUNK, F), jnp.float32),
            pltpu.VMEM((CHUNK, F), jnp.float32),
            pltpu.VMEM_SHARED((n, F), jnp.float32),
            pltpu.SemaphoreType.DMA,
            pltpu.SemaphoreType.DMA,
            pltpu.SemaphoreType.DMA,
            pltpu.SemaphoreType.DMA,
            pltpu.SemaphoreType.DMA,
            pltpu.SemaphoreType.DMA,
        ],
    )
    def k(vp_hbm, z_hbm, src_hbm, dst_hbm, out_hbm, src_v, dst_v,
          rows0, rows1, rows2, acc_sh, g0, g1, g2, s0, s1, s2):
        c = lax.axis_index("c")
        s = lax.axis_index("s")
        wid = c * NS + s
        base = s * rpt
        rows = (rows0, rows1, rows2)
        gsem = (g0, g1, g2)
        ssem = (s0, s1, s2)

        def gather(i, b):
            pltpu.async_copy(vp_hbm.at[src_v.at[i]], rows[b], gsem[b])

        def gwait(i, b):
            pltpu.make_async_copy(vp_hbm.at[src_v.at[i]], rows[b],
                                  gsem[b]).wait()

        def scat(i, b):
            pltpu.async_copy(rows[b], acc_sh.at[dst_v.at[i]], ssem[b],
                             add=True)

        def swait(i, b):
            pltpu.make_async_copy(rows[b], acc_sh.at[dst_v.at[i]],
                                  ssem[b]).wait()

        @pl.when((c == 0) & (s < NIO))
        def _init0():
            pltpu.sync_copy(vp_hbm.at[pl.ds(base, rpt)],
                            acc_sh.at[pl.ds(base, rpt)])

        @pl.when((c == 1) & (s < NIO))
        def _init1():
            pltpu.sync_copy(z_hbm.at[pl.ds(base, rpt)],
                            acc_sh.at[pl.ds(base, rpt)])
        plsc.subcore_barrier()

        def sblock(sb, carry):
            pltpu.sync_copy(src_hbm.at[wid, sb], src_v)
            pltpu.sync_copy(dst_hbm.at[wid, sb], dst_v)
            gather(0, 0)
            gather(1, 1)
            gwait(0, 0)
            scat(0, 0)
            gather(2, 2)

            def group(g, carry2):
                i0 = 3 * g + 1
                for p in range(3):
                    i = i0 + p
                    b = (1 + p) % 3
                    d = p % 3
                    gwait(i, b)
                    scat(i, b)
                    swait(i - 1, d)
                    gather(i + 2, d)
                return carry2

            lax.fori_loop(0, (SBC - 4) // 3, group, 0)
            gwait(SBC - 3, 1)
            scat(SBC - 3, 1)
            swait(SBC - 4, 0)
            gather(SBC - 1, 0)
            gwait(SBC - 2, 2)
            scat(SBC - 2, 2)
            swait(SBC - 3, 1)
            gwait(SBC - 1, 0)
            scat(SBC - 1, 0)
            swait(SBC - 2, 2)
            swait(SBC - 1, 0)
            return carry

        lax.fori_loop(0, SB, sblock, 0)
        plsc.subcore_barrier()

        @pl.when(s < NIO)
        def _writeout():
            pltpu.sync_copy(acc_sh.at[pl.ds(base, rpt)],
                            out_hbm.at[c, pl.ds(base, rpt)])

    return k(vp, zeros, src4, dst4)


def _tc_prescale(degT, x):
    n, F = x.shape
    R = 1000

    def body(degT_ref, x_ref, dinv_ref, xp_ref):
        deg = jnp.sum(degT_ref[...], axis=1, keepdims=True) + 1.0
        dinv = lax.rsqrt(deg)
        dinv_ref[...] = dinv
        xp_ref[...] = x_ref[...] * dinv

    return pl.pallas_call(
        body,
        grid=(n // R,),
        in_specs=[pl.BlockSpec((R, NW), lambda i: (i, 0)),
                  pl.BlockSpec((R, F), lambda i: (i, 0))],
        out_specs=[pl.BlockSpec((R, 1), lambda i: (i, 0)),
                   pl.BlockSpec((R, F), lambda i: (i, 0))],
        out_shape=[jax.ShapeDtypeStruct((n, 1), jnp.float32),
                   jax.ShapeDtypeStruct((n, F), jnp.float32)],
    )(degT, x)


def _tc_mlp(acc, dinv, W1, b1, W2):
    _, n, F = acc.shape
    H = W1.shape[1]
    R = 1000

    def body(a0_ref, a1_ref, dinv_ref, W1_ref, b1_ref, W2_ref, gp_ref):
        agg = (a0_ref[0] + a1_ref[0]) * dinv_ref[...]
        h = jnp.dot(agg, W1_ref[...], preferred_element_type=jnp.float32)
        h = jnp.maximum(h + b1_ref[...], 0.0)
        g = jnp.dot(h, W2_ref[...], preferred_element_type=jnp.float32)
        gp_ref[...] = g * dinv_ref[...]

    return pl.pallas_call(
        body,
        grid=(n // R,),
        in_specs=[pl.BlockSpec((1, R, F), lambda i: (0, i, 0)),
                  pl.BlockSpec((1, R, F), lambda i: (1, i, 0)),
                  pl.BlockSpec((R, 1), lambda i: (i, 0)),
                  pl.BlockSpec((F, H), lambda i: (0, 0)),
                  pl.BlockSpec((1, H), lambda i: (0, 0)),
                  pl.BlockSpec((H, F), lambda i: (0, 0))],
        out_specs=pl.BlockSpec((R, F), lambda i: (i, 0)),
        out_shape=jax.ShapeDtypeStruct((n, F), jnp.float32),
    )(acc, acc, dinv, W1, b1, W2)


def _tc_final(acc, dinv, b2):
    _, n, F = acc.shape
    R = 1000

    def body(c0_ref, c1_ref, dinv_ref, b2_ref, out_ref):
        agg = (c0_ref[0] + c1_ref[0]) * dinv_ref[...]
        out_ref[...] = agg + b2_ref[...]

    return pl.pallas_call(
        body,
        grid=(n // R,),
        in_specs=[pl.BlockSpec((1, R, F), lambda i: (0, i, 0)),
                  pl.BlockSpec((1, R, F), lambda i: (1, i, 0)),
                  pl.BlockSpec((R, 1), lambda i: (i, 0)),
                  pl.BlockSpec((1, F), lambda i: (0, 0))],
        out_specs=pl.BlockSpec((R, F), lambda i: (i, 0)),
        out_shape=jax.ShapeDtypeStruct((n, F), jnp.float32),
    )(acc, acc, dinv, b2)


def kernel(x, edge_index, W1, b1, W2, b2):
    n, F = x.shape
    src4 = edge_index[0].reshape(NW, SB, SBC, CHUNK)
    dst4 = edge_index[1].reshape(NW, SB, SBC, CHUNK)
    zeros = jnp.zeros((n, F), jnp.float32)

    degp = _sc_degree(dst4, n)
    dinv, xp = _tc_prescale(degp.T, x)
    acc1 = _sc_aggregate(xp, zeros, src4, dst4)
    gp = _tc_mlp(acc1, dinv, W1, b1.reshape(1, -1), W2)
    acc2 = _sc_aggregate(gp, zeros, src4, dst4)
    out = _tc_final(acc2, dinv, b2.reshape(1, -1))
    return out


# speedup vs baseline: 37.8914x; 1.0283x over previous
import functools

import jax
import jax.numpy as jnp
from jax import lax
from jax.experimental import pallas as pl
from jax.experimental.pallas import tpu as pltpu
from jax.experimental.pallas import tpu_sc as plsc

NC = 2
NS = 16
NW = NC * NS
CHUNK = 80
SB = 5
SBC = 25


def _sc_degree(dst4, n):
    mesh = plsc.VectorSubcoreMesh(core_axis_name="c", subcore_axis_name="s",
                                  num_cores=NC, num_subcores=NS)

    @functools.partial(
        pl.kernel,
        out_type=jax.ShapeDtypeStruct((NW, n), jnp.float32),
        mesh=mesh,
        scratch_types=[
            pltpu.VMEM((SBC, CHUNK), jnp.int32),
            pltpu.VMEM((n,), jnp.float32),
        ],
        compiler_params=pltpu.CompilerParams(needs_layout_passes=False),
    )
    def k(dst_hbm, out_hbm, dst_v, deg_v):
        c = lax.axis_index("c")
        s = lax.axis_index("s")
        wid = c * NS + s

        zeros16 = jnp.zeros((16,), jnp.float32)

        def zbody(i, carry):
            deg_v[pl.ds(i * 16, 16)] = zeros16
            return carry

        lax.fori_loop(0, n // 16, zbody, 0)

        ones16 = jnp.ones((16,), jnp.float32)

        def sblock(sb, carry):
            pltpu.sync_copy(dst_hbm.at[wid, sb], dst_v)

            def hbody(i, carry2):
                for j in range(CHUNK // 16):
                    idx = dst_v[i, pl.ds(j * 16, 16)]
                    plsc.addupdate_scatter(deg_v, [idx], ones16)
                return carry2

            lax.fori_loop(0, SBC, hbody, 0)
            return carry

        lax.fori_loop(0, SB, sblock, 0)
        pltpu.sync_copy(deg_v, out_hbm.at[wid])

    return k(dst4)


def _sc_aggregate(vp, zeros, src1, dst4):
    n, F = vp.shape
    NIO = 10
    rpt = n // NIO
    mesh = plsc.VectorSubcoreMesh(core_axis_name="c", subcore_axis_name="s",
                                  num_cores=NC, num_subcores=NS)

    @functools.partial(
        pl.kernel,
        out_type=jax.ShapeDtypeStruct((NC, n, F), jnp.float32),
        mesh=mesh,
        scratch_types=[
            pltpu.VMEM((SBC * CHUNK,), jnp.int32),
            pltpu.VMEM((SBC, CHUNK), jnp.int32),
            pltpu.VMEM((CHUNK, F), jnp.float32),
            pltpu.VMEM((CHUNK, F), jnp.float32),
            pltpu.VMEM((CHUNK, F), jnp.float32),
            pltpu.VMEM_SHARED((n, F), jnp.float32),
            pltpu.SemaphoreType.DMA,
            pltpu.SemaphoreType.DMA,
            pltpu.SemaphoreType.DMA,
            pltpu.SemaphoreType.DMA,
            pltpu.SemaphoreType.DMA,
            pltpu.SemaphoreType.DMA,
        ],
    )
    def k(vp_hbm, z_hbm, src_hbm, dst_hbm, out_hbm, src_v, dst_v,
          rows0, rows1, rows2, acc_sh, g0, g1, g2, s0, s1, s2):
        c = lax.axis_index("c")
        s = lax.axis_index("s")
        wid = c * NS + s
        base = s * rpt
        rows = (rows0, rows1, rows2)
        gsem = (g0, g1, g2)
        ssem = (s0, s1, s2)

        def gather(i, b):
            pltpu.async_copy(vp_hbm.at[src_v.at[pl.ds(i * CHUNK, CHUNK)]],
                             rows[b], gsem[b])

        def gwait(i, b):
            pltpu.make_async_copy(
                vp_hbm.at[src_v.at[pl.ds(i * CHUNK, CHUNK)]], rows[b],
                gsem[b]).wait()

        def scat(i, b):
            pltpu.async_copy(rows[b], acc_sh.at[dst_v.at[i]], ssem[b],
                             add=True)

        def swait(i, b):
            pltpu.make_async_copy(rows[b], acc_sh.at[dst_v.at[i]],
                                  ssem[b]).wait()

        @pl.when((c == 0) & (s < NIO))
        def _init0():
            pltpu.sync_copy(vp_hbm.at[pl.ds(base, rpt)],
                            acc_sh.at[pl.ds(base, rpt)])

        @pl.when((c == 1) & (s < NIO))
        def _init1():
            pltpu.sync_copy(z_hbm.at[pl.ds(base, rpt)],
                            acc_sh.at[pl.ds(base, rpt)])
        plsc.subcore_barrier()

        def sblock(sb, carry):
            pltpu.sync_copy(
                src_hbm.at[pl.ds((wid * SB + sb) * SBC * CHUNK, SBC * CHUNK)],
                src_v)
            pltpu.sync_copy(dst_hbm.at[wid, sb], dst_v)
            gather(0, 0)
            gather(1, 1)
            gwait(0, 0)
            scat(0, 0)
            gather(2, 2)

            def group(g, carry2):
                i0 = 3 * g + 1
                for p in range(3):
                    i = i0 + p
                    b = (1 + p) % 3
                    d = p % 3
                    gwait(i, b)
                    scat(i, b)
                    swait(i - 1, d)
                    gather(i + 2, d)
                return carry2

            lax.fori_loop(0, (SBC - 4) // 3, group, 0)
            gwait(SBC - 3, 1)
            scat(SBC - 3, 1)
            swait(SBC - 4, 0)
            gather(SBC - 1, 0)
            gwait(SBC - 2, 2)
            scat(SBC - 2, 2)
            swait(SBC - 3, 1)
            gwait(SBC - 1, 0)
            scat(SBC - 1, 0)
            swait(SBC - 2, 2)
            swait(SBC - 1, 0)
            return carry

        lax.fori_loop(0, SB, sblock, 0)
        plsc.subcore_barrier()

        @pl.when(s < NIO)
        def _writeout():
            pltpu.sync_copy(acc_sh.at[pl.ds(base, rpt)],
                            out_hbm.at[c, pl.ds(base, rpt)])

    return k(vp, zeros, src1, dst4)


def _tc_prescale(degp, x):
    n, F = x.shape
    R = 1000

    def body(degp_ref, x_ref, dinv_ref, xp_ref):
        deg = jnp.sum(degp_ref[...], axis=0, keepdims=True) + 1.0
        dinv = jnp.transpose(lax.rsqrt(deg))
        dinv_ref[...] = dinv
        xp_ref[...] = x_ref[...] * dinv

    return pl.pallas_call(
        body,
        out_shape=[jax.ShapeDtypeStruct((n, 1), jnp.float32),
                   jax.ShapeDtypeStruct((n, F), jnp.float32)],
    )(degp, x)


def _tc_mlp(acc, dinv, W1, b1, W2):
    _, n, F = acc.shape
    H = W1.shape[1]
    R = 1000

    def body(a0_ref, a1_ref, dinv_ref, W1_ref, b1_ref, W2_ref, gp_ref):
        agg = (a0_ref[0] + a1_ref[0]) * dinv_ref[...]
        h = jnp.dot(agg, W1_ref[...], preferred_element_type=jnp.float32)
        h = jnp.maximum(h + b1_ref[...], 0.0)
        g = jnp.dot(h, W2_ref[...], preferred_element_type=jnp.float32)
        gp_ref[...] = g * dinv_ref[...]

    return pl.pallas_call(
        body,
        grid=(n // R,),
        in_specs=[pl.BlockSpec((1, R, F), lambda i: (0, i, 0)),
                  pl.BlockSpec((1, R, F), lambda i: (1, i, 0)),
                  pl.BlockSpec((R, 1), lambda i: (i, 0)),
                  pl.BlockSpec((F, H), lambda i: (0, 0)),
                  pl.BlockSpec((1, H), lambda i: (0, 0)),
                  pl.BlockSpec((H, F), lambda i: (0, 0))],
        out_specs=pl.BlockSpec((R, F), lambda i: (i, 0)),
        out_shape=jax.ShapeDtypeStruct((n, F), jnp.float32),
    )(acc, acc, dinv, W1, b1, W2)


def _tc_final(acc, dinv, b2):
    _, n, F = acc.shape
    R = 1000

    def body(c0_ref, c1_ref, dinv_ref, b2_ref, out_ref):
        agg = (c0_ref[0] + c1_ref[0]) * dinv_ref[...]
        out_ref[...] = agg + b2_ref[...]

    return pl.pallas_call(
        body,
        grid=(n // R,),
        in_specs=[pl.BlockSpec((1, R, F), lambda i: (0, i, 0)),
                  pl.BlockSpec((1, R, F), lambda i: (1, i, 0)),
                  pl.BlockSpec((R, 1), lambda i: (i, 0)),
                  pl.BlockSpec((1, F), lambda i: (0, 0))],
        out_specs=pl.BlockSpec((R, F), lambda i: (i, 0)),
        out_shape=jax.ShapeDtypeStruct((n, F), jnp.float32),
    )(acc, acc, dinv, b2)


def kernel(x, edge_index, W1, b1, W2, b2):
    n, F = x.shape
    src1 = edge_index[0]
    dst4 = edge_index[1].reshape(NW, SB, SBC, CHUNK)
    zeros = jnp.zeros((n, F), jnp.float32)

    degp = _sc_degree(dst4, n)
    dinv, xp = _tc_prescale(degp, x)
    acc1 = _sc_aggregate(xp, zeros, src1, dst4)
    gp = _tc_mlp(acc1, dinv, W1, b1.reshape(1, -1), W2)
    acc2 = _sc_aggregate(gp, zeros, src1, dst4)
    out = _tc_final(acc2, dinv, b2.reshape(1, -1))
    return out


# speedup vs baseline: 39.7987x; 1.0503x over previous
import functools

import jax
import jax.numpy as jnp
from jax import lax
from jax.experimental import pallas as pl
from jax.experimental.pallas import tpu as pltpu
from jax.experimental.pallas import tpu_sc as plsc

NC = 2
NS = 16
NW = NC * NS
CHUNK = 80
SB = 5
SBC = 25


def _sc_degree(dst4, n):
    mesh = plsc.VectorSubcoreMesh(core_axis_name="c", subcore_axis_name="s",
                                  num_cores=NC, num_subcores=NS)

    @functools.partial(
        pl.kernel,
        out_type=jax.ShapeDtypeStruct((NW, n), jnp.float32),
        mesh=mesh,
        scratch_types=[
            pltpu.VMEM((SBC, CHUNK), jnp.int32),
            pltpu.VMEM((n,), jnp.float32),
        ],
        compiler_params=pltpu.CompilerParams(needs_layout_passes=False),
    )
    def k(dst_hbm, out_hbm, dst_v, deg_v):
        c = lax.axis_index("c")
        s = lax.axis_index("s")
        wid = c * NS + s

        zeros16 = jnp.zeros((16,), jnp.float32)

        def zbody(i, carry):
            deg_v[pl.ds(i * 16, 16)] = zeros16
            return carry

        lax.fori_loop(0, n // 16, zbody, 0)

        ones16 = jnp.ones((16,), jnp.float32)

        def sblock(sb, carry):
            pltpu.sync_copy(dst_hbm.at[wid, sb], dst_v)

            def hbody(i, carry2):
                for j in range(CHUNK // 16):
                    idx = dst_v[i, pl.ds(j * 16, 16)]
                    plsc.addupdate_scatter(deg_v, [idx], ones16)
                return carry2

            lax.fori_loop(0, SBC, hbody, 0)
            return carry

        lax.fori_loop(0, SB, sblock, 0)
        pltpu.sync_copy(deg_v, out_hbm.at[wid])

    return k(dst4)


def _sc_aggregate(vp, zeros, src1, dst4):
    n, F = vp.shape
    NIO = 10
    rpt = n // NIO
    mesh = plsc.VectorSubcoreMesh(core_axis_name="c", subcore_axis_name="s",
                                  num_cores=NC, num_subcores=NS)

    @functools.partial(
        pl.kernel,
        out_type=jax.ShapeDtypeStruct((NC, n, F), jnp.float32),
        mesh=mesh,
        scratch_types=[
            pltpu.VMEM((SBC * CHUNK,), jnp.int32),
            pltpu.VMEM((SBC * CHUNK,), jnp.int32),
            pltpu.VMEM((SBC, CHUNK), jnp.int32),
            pltpu.VMEM((CHUNK, F), jnp.float32),
            pltpu.VMEM((CHUNK, F), jnp.float32),
            pltpu.VMEM((CHUNK, F), jnp.float32),
            pltpu.VMEM_SHARED((n, F), jnp.float32),
            pltpu.SemaphoreType.DMA,
            pltpu.SemaphoreType.DMA,
            pltpu.SemaphoreType.DMA,
            pltpu.SemaphoreType.DMA,
            pltpu.SemaphoreType.DMA,
            pltpu.SemaphoreType.DMA,
            pltpu.SemaphoreType.DMA,
            pltpu.SemaphoreType.DMA,
        ],
    )
    def k(vp_hbm, z_hbm, src_hbm, dst_hbm, out_hbm, srcA, srcB, dst_v,
          rows0, rows1, rows2, acc_sh, g0, g1, g2, s0, s1, s2, iA, iB):
        c = lax.axis_index("c")
        s = lax.axis_index("s")
        wid = c * NS + s
        base = s * rpt
        ebase = wid * SB * SBC * CHUNK
        rows = (rows0, rows1, rows2)
        gsem = (g0, g1, g2)
        ssem = (s0, s1, s2)
        srcbuf = (srcA, srcB)
        isem = (iA, iB)
        nch = SB * SBC

        def sload(m, sync):
            sv = srcbuf[m % 2]
            hs = src_hbm.at[pl.ds(ebase + m * SBC * CHUNK, SBC * CHUNK)]
            if sync:
                pltpu.sync_copy(hs, sv)
            else:
                pltpu.async_copy(hs, sv, isem[m % 2])

        def swaitload(m):
            sv = srcbuf[m % 2]
            hs = src_hbm.at[pl.ds(ebase + m * SBC * CHUNK, SBC * CHUNK)]
            pltpu.make_async_copy(hs, sv, isem[m % 2]).wait()

        def gref(j):
            sv = srcbuf[(j // SBC) % 2]
            return vp_hbm.at[sv.at[pl.ds((j % SBC) * CHUNK, CHUNK)]]

        def gather(j):
            pltpu.async_copy(gref(j), rows[j % 3], gsem[j % 3])

        def gwait(j):
            pltpu.make_async_copy(gref(j), rows[j % 3], gsem[j % 3]).wait()

        def scat(j):
            pltpu.async_copy(rows[j % 3], acc_sh.at[dst_v.at[j % SBC]],
                             ssem[j % 3], add=True)

        def swait(j):
            pltpu.make_async_copy(rows[j % 3], acc_sh.at[dst_v.at[j % SBC]],
                                  ssem[j % 3]).wait()

        sload(0, True)
        sload(1, True)
        pltpu.sync_copy(dst_hbm.at[wid, 0], dst_v)
        gather(0)
        gather(1)

        @pl.when((c == 0) & (s < NIO))
        def _init0():
            pltpu.sync_copy(vp_hbm.at[pl.ds(base, rpt)],
                            acc_sh.at[pl.ds(base, rpt)])

        @pl.when((c == 1) & (s < NIO))
        def _init1():
            pltpu.sync_copy(z_hbm.at[pl.ds(base, rpt)],
                            acc_sh.at[pl.ds(base, rpt)])
        plsc.subcore_barrier()

        for j in range(nch):
            p, m = j % SBC, j // SBC
            if j == 0:
                gwait(0)
                scat(0)
                gather(2)
                continue
            gwait(j)
            if p == SBC - 1 and m + 2 < SB:
                sload(m + 2, False)
            if p == 0:
                swait(j - 1)
                pltpu.sync_copy(dst_hbm.at[wid, m], dst_v)
                scat(j)
            else:
                scat(j)
                swait(j - 1)
            if j + 2 < nch:
                t = j + 2
                if t % SBC == 0 and t // SBC >= 2:
                    swaitload(t // SBC)
                gather(t)
        swait(nch - 1)
        plsc.subcore_barrier()

        @pl.when(s < NIO)
        def _writeout():
            pltpu.sync_copy(acc_sh.at[pl.ds(base, rpt)],
                            out_hbm.at[c, pl.ds(base, rpt)])

    return k(vp, zeros, src1, dst4)


def _tc_prescale(degp, x):
    n, F = x.shape
    R = 1000

    def body(degp_ref, x_ref, dinv_ref, xp_ref):
        deg = jnp.sum(degp_ref[...], axis=0, keepdims=True) + 1.0
        dinv = jnp.transpose(lax.rsqrt(deg))
        dinv_ref[...] = dinv
        xp_ref[...] = x_ref[...] * dinv

    return pl.pallas_call(
        body,
        out_shape=[jax.ShapeDtypeStruct((n, 1), jnp.float32),
                   jax.ShapeDtypeStruct((n, F), jnp.float32)],
    )(degp, x)


def _tc_mlp(acc, dinv, W1, b1, W2):
    _, n, F = acc.shape
    H = W1.shape[1]
    R = 1000

    def body(a0_ref, a1_ref, dinv_ref, W1_ref, b1_ref, W2_ref, gp_ref):
        agg = (a0_ref[0] + a1_ref[0]) * dinv_ref[...]
        h = jnp.dot(agg, W1_ref[...], preferred_element_type=jnp.float32)
        h = jnp.maximum(h + b1_ref[...], 0.0)
        g = jnp.dot(h, W2_ref[...], preferred_element_type=jnp.float32)
        gp_ref[...] = g * dinv_ref[...]

    return pl.pallas_call(
        body,
        grid=(n // R,),
        in_specs=[pl.BlockSpec((1, R, F), lambda i: (0, i, 0)),
                  pl.BlockSpec((1, R, F), lambda i: (1, i, 0)),
                  pl.BlockSpec((R, 1), lambda i: (i, 0)),
                  pl.BlockSpec((F, H), lambda i: (0, 0)),
                  pl.BlockSpec((1, H), lambda i: (0, 0)),
                  pl.BlockSpec((H, F), lambda i: (0, 0))],
        out_specs=pl.BlockSpec((R, F), lambda i: (i, 0)),
        out_shape=jax.ShapeDtypeStruct((n, F), jnp.float32),
    )(acc, acc, dinv, W1, b1, W2)


def _tc_final(acc, dinv, b2):
    _, n, F = acc.shape
    R = 1000

    def body(c0_ref, c1_ref, dinv_ref, b2_ref, out_ref):
        agg = (c0_ref[0] + c1_ref[0]) * dinv_ref[...]
        out_ref[...] = agg + b2_ref[...]

    return pl.pallas_call(
        body,
        grid=(n // R,),
        in_specs=[pl.BlockSpec((1, R, F), lambda i: (0, i, 0)),
                  pl.BlockSpec((1, R, F), lambda i: (1, i, 0)),
                  pl.BlockSpec((R, 1), lambda i: (i, 0)),
                  pl.BlockSpec((1, F), lambda i: (0, 0))],
        out_specs=pl.BlockSpec((R, F), lambda i: (i, 0)),
        out_shape=jax.ShapeDtypeStruct((n, F), jnp.float32),
    )(acc, acc, dinv, b2)


def kernel(x, edge_index, W1, b1, W2, b2):
    n, F = x.shape
    src1 = edge_index[0]
    dst4 = edge_index[1].reshape(NW, SB, SBC, CHUNK)
    zeros = jnp.zeros((n, F), jnp.float32)

    degp = _sc_degree(dst4, n)
    dinv, xp = _tc_prescale(degp, x)
    acc1 = _sc_aggregate(xp, zeros, src1, dst4)
    gp = _tc_mlp(acc1, dinv, W1, b1.reshape(1, -1), W2)
    acc2 = _sc_aggregate(gp, zeros, src1, dst4)
    out = _tc_final(acc2, dinv, b2.reshape(1, -1))
    return out


# speedup vs baseline: 39.8760x; 1.0019x over previous
import functools

import jax
import jax.numpy as jnp
from jax import lax
from jax.experimental import pallas as pl
from jax.experimental.pallas import tpu as pltpu
from jax.experimental.pallas import tpu_sc as plsc

NC = 2
NS = 16
NW = NC * NS
CHUNK = 80
SB = 5
SBC = 25


def _sc_degree(dst4, n):
    mesh = plsc.VectorSubcoreMesh(core_axis_name="c", subcore_axis_name="s",
                                  num_cores=NC, num_subcores=NS)

    @functools.partial(
        pl.kernel,
        out_type=jax.ShapeDtypeStruct((NW, n), jnp.float32),
        mesh=mesh,
        scratch_types=[
            pltpu.VMEM((SBC, CHUNK), jnp.int32),
            pltpu.VMEM((n,), jnp.float32),
        ],
        compiler_params=pltpu.CompilerParams(needs_layout_passes=False),
    )
    def k(dst_hbm, out_hbm, dst_v, deg_v):
        c = lax.axis_index("c")
        s = lax.axis_index("s")
        wid = c * NS + s

        zeros16 = jnp.zeros((16,), jnp.float32)

        def zbody(i, carry):
            deg_v[pl.ds(i * 16, 16)] = zeros16
            return carry

        lax.fori_loop(0, n // 16, zbody, 0)

        ones16 = jnp.ones((16,), jnp.float32)

        def sblock(sb, carry):
            pltpu.sync_copy(dst_hbm.at[wid, sb], dst_v)

            def hbody(i, carry2):
                for j in range(CHUNK // 16):
                    idx = dst_v[i, pl.ds(j * 16, 16)]
                    plsc.addupdate_scatter(deg_v, [idx], ones16)
                return carry2

            lax.fori_loop(0, SBC, hbody, 0)
            return carry

        lax.fori_loop(0, SB, sblock, 0)
        pltpu.sync_copy(deg_v, out_hbm.at[wid])

    return k(dst4)


def _sc_aggregate(vp, zeros, src1, dst4):
    n, F = vp.shape
    RPT = 624
    RLAST = n - RPT * (NS - 1)
    mesh = plsc.VectorSubcoreMesh(core_axis_name="c", subcore_axis_name="s",
                                  num_cores=NC, num_subcores=NS)

    @functools.partial(
        pl.kernel,
        out_type=jax.ShapeDtypeStruct((NC, n, F), jnp.float32),
        mesh=mesh,
        scratch_types=[
            pltpu.VMEM((SBC * CHUNK,), jnp.int32),
            pltpu.VMEM((SBC * CHUNK,), jnp.int32),
            pltpu.VMEM((SBC, CHUNK), jnp.int32),
            pltpu.VMEM((CHUNK, F), jnp.float32),
            pltpu.VMEM((CHUNK, F), jnp.float32),
            pltpu.VMEM((CHUNK, F), jnp.float32),
            pltpu.VMEM_SHARED((n, F), jnp.float32),
            pltpu.SemaphoreType.DMA,
            pltpu.SemaphoreType.DMA,
            pltpu.SemaphoreType.DMA,
            pltpu.SemaphoreType.DMA,
            pltpu.SemaphoreType.DMA,
            pltpu.SemaphoreType.DMA,
            pltpu.SemaphoreType.DMA,
            pltpu.SemaphoreType.DMA,
        ],
    )
    def k(vp_hbm, z_hbm, src_hbm, dst_hbm, out_hbm, srcA, srcB, dst_v,
          rows0, rows1, rows2, acc_sh, g0, g1, g2, s0, s1, s2, iA, iB):
        c = lax.axis_index("c")
        s = lax.axis_index("s")
        wid = c * NS + s
        base = s * RPT
        ebase = wid * SB * SBC * CHUNK
        rows = (rows0, rows1, rows2)
        gsem = (g0, g1, g2)
        ssem = (s0, s1, s2)
        srcbuf = (srcA, srcB)
        isem = (iA, iB)
        nch = SB * SBC

        def sload(m, sync):
            sv = srcbuf[m % 2]
            hs = src_hbm.at[pl.ds(ebase + m * SBC * CHUNK, SBC * CHUNK)]
            if sync:
                pltpu.sync_copy(hs, sv)
            else:
                pltpu.async_copy(hs, sv, isem[m % 2])

        def swaitload(m):
            sv = srcbuf[m % 2]
            hs = src_hbm.at[pl.ds(ebase + m * SBC * CHUNK, SBC * CHUNK)]
            pltpu.make_async_copy(hs, sv, isem[m % 2]).wait()

        def gref(j):
            sv = srcbuf[(j // SBC) % 2]
            return vp_hbm.at[sv.at[pl.ds((j % SBC) * CHUNK, CHUNK)]]

        def gather(j):
            pltpu.async_copy(gref(j), rows[j % 3], gsem[j % 3])

        def gwait(j):
            pltpu.make_async_copy(gref(j), rows[j % 3], gsem[j % 3]).wait()

        def scat(j):
            pltpu.async_copy(rows[j % 3], acc_sh.at[dst_v.at[j % SBC]],
                             ssem[j % 3], add=True)

        def swait(j):
            pltpu.make_async_copy(rows[j % 3], acc_sh.at[dst_v.at[j % SBC]],
                                  ssem[j % 3]).wait()

        sload(0, True)
        sload(1, True)
        pltpu.sync_copy(dst_hbm.at[wid, 0], dst_v)
        gather(0)
        gather(1)

        init_src = (vp_hbm, z_hbm)
        for cc in range(NC):
            @pl.when((c == cc) & (s < NS - 1))
            def _init_a(cc=cc):
                pltpu.sync_copy(init_src[cc].at[pl.ds(base, RPT)],
                                acc_sh.at[pl.ds(base, RPT)])

            @pl.when((c == cc) & (s == NS - 1))
            def _init_b(cc=cc):
                pltpu.sync_copy(init_src[cc].at[pl.ds(base, RLAST)],
                                acc_sh.at[pl.ds(base, RLAST)])
        plsc.subcore_barrier()

        for j in range(nch):
            p, m = j % SBC, j // SBC
            if j == 0:
                gwait(0)
                scat(0)
                gather(2)
                continue
            gwait(j)
            if p == SBC - 1 and m + 2 < SB:
                sload(m + 2, False)
            if p == 0:
                swait(j - 1)
                pltpu.sync_copy(dst_hbm.at[wid, m], dst_v)
                scat(j)
            else:
                scat(j)
                swait(j - 1)
            if j + 2 < nch:
                t = j + 2
                if t % SBC == 0 and t // SBC >= 2:
                    swaitload(t // SBC)
                gather(t)
        swait(nch - 1)
        plsc.subcore_barrier()

        @pl.when(s < NS - 1)
        def _writeout_a():
            pltpu.sync_copy(acc_sh.at[pl.ds(base, RPT)],
                            out_hbm.at[c, pl.ds(base, RPT)])

        @pl.when(s == NS - 1)
        def _writeout_b():
            pltpu.sync_copy(acc_sh.at[pl.ds(base, RLAST)],
                            out_hbm.at[c, pl.ds(base, RLAST)])

    return k(vp, zeros, src1, dst4)


def _tc_prescale(degp, x):
    n, F = x.shape
    R = 1000

    def body(degp_ref, x_ref, dinv_ref, xp_ref):
        deg = jnp.sum(degp_ref[...], axis=0, keepdims=True) + 1.0
        dinv = jnp.transpose(lax.rsqrt(deg))
        dinv_ref[...] = dinv
        xp_ref[...] = x_ref[...] * dinv

    return pl.pallas_call(
        body,
        out_shape=[jax.ShapeDtypeStruct((n, 1), jnp.float32),
                   jax.ShapeDtypeStruct((n, F), jnp.float32)],
    )(degp, x)


def _tc_mlp(acc, dinv, W1, b1, W2):
    _, n, F = acc.shape
    H = W1.shape[1]
    R = 1000

    def body(a0_ref, a1_ref, dinv_ref, W1_ref, b1_ref, W2_ref, gp_ref):
        agg = (a0_ref[0] + a1_ref[0]) * dinv_ref[...]
        h = jnp.dot(agg, W1_ref[...], preferred_element_type=jnp.float32)
        h = jnp.maximum(h + b1_ref[...], 0.0)
        g = jnp.dot(h, W2_ref[...], preferred_element_type=jnp.float32)
        gp_ref[...] = g * dinv_ref[...]

    return pl.pallas_call(
        body,
        grid=(n // R,),
        in_specs=[pl.BlockSpec((1, R, F), lambda i: (0, i, 0)),
                  pl.BlockSpec((1, R, F), lambda i: (1, i, 0)),
                  pl.BlockSpec((R, 1), lambda i: (i, 0)),
                  pl.BlockSpec((F, H), lambda i: (0, 0)),
                  pl.BlockSpec((1, H), lambda i: (0, 0)),
                  pl.BlockSpec((H, F), lambda i: (0, 0))],
        out_specs=pl.BlockSpec((R, F), lambda i: (i, 0)),
        out_shape=jax.ShapeDtypeStruct((n, F), jnp.float32),
    )(acc, acc, dinv, W1, b1, W2)


def _tc_final(acc, dinv, b2):
    _, n, F = acc.shape
    R = 1000

    def body(c0_ref, c1_ref, dinv_ref, b2_ref, out_ref):
        agg = (c0_ref[0] + c1_ref[0]) * dinv_ref[...]
        out_ref[...] = agg + b2_ref[...]

    return pl.pallas_call(
        body,
        grid=(n // R,),
        in_specs=[pl.BlockSpec((1, R, F), lambda i: (0, i, 0)),
                  pl.BlockSpec((1, R, F), lambda i: (1, i, 0)),
                  pl.BlockSpec((R, 1), lambda i: (i, 0)),
                  pl.BlockSpec((1, F), lambda i: (0, 0))],
        out_specs=pl.BlockSpec((R, F), lambda i: (i, 0)),
        out_shape=jax.ShapeDtypeStruct((n, F), jnp.float32),
    )(acc, acc, dinv, b2)


def kernel(x, edge_index, W1, b1, W2, b2):
    n, F = x.shape
    src1 = edge_index[0]
    dst4 = edge_index[1].reshape(NW, SB, SBC, CHUNK)
    zeros = jnp.zeros((n, F), jnp.float32)

    degp = _sc_degree(dst4, n)
    dinv, xp = _tc_prescale(degp, x)
    acc1 = _sc_aggregate(xp, zeros, src1, dst4)
    gp = _tc_mlp(acc1, dinv, W1, b1.reshape(1, -1), W2)
    acc2 = _sc_aggregate(gp, zeros, src1, dst4)
    out = _tc_final(acc2, dinv, b2.reshape(1, -1))
    return out


# speedup vs baseline: 40.6053x; 1.0183x over previous
import functools

import jax
import jax.numpy as jnp
from jax import lax
from jax.experimental import pallas as pl
from jax.experimental.pallas import tpu as pltpu
from jax.experimental.pallas import tpu_sc as plsc

NC = 2
NS = 16
NW = NC * NS
CHUNK = 80
SB = 5
SBC = 25


def _sc_degree(dst4, n):
    mesh = plsc.VectorSubcoreMesh(core_axis_name="c", subcore_axis_name="s",
                                  num_cores=NC, num_subcores=NS)

    @functools.partial(
        pl.kernel,
        out_type=jax.ShapeDtypeStruct((NW, n), jnp.float32),
        mesh=mesh,
        scratch_types=[
            pltpu.VMEM((SBC, CHUNK), jnp.int32),
            pltpu.VMEM((n,), jnp.float32),
        ],
        compiler_params=pltpu.CompilerParams(needs_layout_passes=False),
    )
    def k(dst_hbm, out_hbm, dst_v, deg_v):
        c = lax.axis_index("c")
        s = lax.axis_index("s")
        wid = c * NS + s

        zeros16 = jnp.zeros((16,), jnp.float32)

        def zbody(i, carry):
            deg_v[pl.ds(i * 16, 16)] = zeros16
            return carry

        lax.fori_loop(0, n // 16, zbody, 0)

        ones16 = jnp.ones((16,), jnp.float32)

        def sblock(sb, carry):
            pltpu.sync_copy(dst_hbm.at[wid, sb], dst_v)

            def hbody(i, carry2):
                for j in range(CHUNK // 16):
                    idx = dst_v[i, pl.ds(j * 16, 16)]
                    plsc.addupdate_scatter(deg_v, [idx], ones16)
                return carry2

            lax.fori_loop(0, SBC, hbody, 0)
            return carry

        lax.fori_loop(0, SB, sblock, 0)
        pltpu.sync_copy(deg_v, out_hbm.at[wid])

    return k(dst4)


def _sc_aggregate(vp, zeros, src1, dst4):
    n, F = vp.shape
    RPT = 624
    RLAST = n - RPT * (NS - 1)
    mesh = plsc.VectorSubcoreMesh(core_axis_name="c", subcore_axis_name="s",
                                  num_cores=NC, num_subcores=NS)

    @functools.partial(
        pl.kernel,
        out_type=jax.ShapeDtypeStruct((NC, n, F), jnp.float32),
        mesh=mesh,
        scratch_types=[
            pltpu.VMEM((SBC * CHUNK,), jnp.int32),
            pltpu.VMEM((SBC * CHUNK,), jnp.int32),
            pltpu.VMEM((SBC, CHUNK), jnp.int32),
            pltpu.VMEM((CHUNK, F), jnp.float32),
            pltpu.VMEM((CHUNK, F), jnp.float32),
            pltpu.VMEM((CHUNK, F), jnp.float32),
            pltpu.VMEM_SHARED((n, F), jnp.float32),
            pltpu.SemaphoreType.DMA,
            pltpu.SemaphoreType.DMA,
            pltpu.SemaphoreType.DMA,
            pltpu.SemaphoreType.DMA,
            pltpu.SemaphoreType.DMA,
            pltpu.SemaphoreType.DMA,
            pltpu.SemaphoreType.DMA,
            pltpu.SemaphoreType.DMA,
        ],
    )
    def k(vp_hbm, z_hbm, src_hbm, dst_hbm, out_hbm, srcA, srcB, dst_v,
          rows0, rows1, rows2, acc_sh, g0, g1, g2, s0, s1, s2, iA, iB):
        c = lax.axis_index("c")
        s = lax.axis_index("s")
        wid = c * NS + s
        base = s * RPT
        ebase = wid * SB * SBC * CHUNK
        rows = (rows0, rows1, rows2)
        gsem = (g0, g1, g2)
        ssem = (s0, s1, s2)
        srcbuf = (srcA, srcB)
        isem = (iA, iB)
        nch = SB * SBC

        def sload(m, sync):
            sv = srcbuf[m % 2]
            hs = src_hbm.at[pl.ds(ebase + m * SBC * CHUNK, SBC * CHUNK)]
            if sync:
                pltpu.sync_copy(hs, sv)
            else:
                pltpu.async_copy(hs, sv, isem[m % 2])

        def swaitload(m):
            sv = srcbuf[m % 2]
            hs = src_hbm.at[pl.ds(ebase + m * SBC * CHUNK, SBC * CHUNK)]
            pltpu.make_async_copy(hs, sv, isem[m % 2]).wait()

        def gref(j):
            sv = srcbuf[(j // SBC) % 2]
            return vp_hbm.at[sv.at[pl.ds((j % SBC) * CHUNK, CHUNK)]]

        def gather(j):
            pltpu.async_copy(gref(j), rows[j % 3], gsem[j % 3])

        def gwait(j):
            pltpu.make_async_copy(gref(j), rows[j % 3], gsem[j % 3]).wait()

        def scat(j):
            pltpu.async_copy(rows[j % 3], acc_sh.at[dst_v.at[j % SBC]],
                             ssem[j % 3], add=True)

        def swait(j):
            pltpu.make_async_copy(rows[j % 3], acc_sh.at[dst_v.at[j % SBC]],
                                  ssem[j % 3]).wait()

        sload(0, True)
        sload(1, True)
        pltpu.sync_copy(dst_hbm.at[wid, 0], dst_v)
        gather(0)
        gather(1)

        init_src = (vp_hbm, z_hbm)
        for cc in range(NC):
            @pl.when((c == cc) & (s < NS - 1))
            def _init_a(cc=cc):
                pltpu.sync_copy(init_src[cc].at[pl.ds(base, RPT)],
                                acc_sh.at[pl.ds(base, RPT)])

            @pl.when((c == cc) & (s == NS - 1))
            def _init_b(cc=cc):
                pltpu.sync_copy(init_src[cc].at[pl.ds(base, RLAST)],
                                acc_sh.at[pl.ds(base, RLAST)])
        plsc.subcore_barrier()

        for j in range(nch):
            p, m = j % SBC, j // SBC
            if j == 0:
                gwait(0)
                scat(0)
                gather(2)
                continue
            gwait(j)
            if p == SBC - 1 and m + 2 < SB:
                sload(m + 2, False)
            if p == 0:
                swait(j - 1)
                pltpu.sync_copy(dst_hbm.at[wid, m], dst_v)
                scat(j)
            else:
                scat(j)
                swait(j - 1)
            if j + 2 < nch:
                t = j + 2
                if t % SBC == 0 and t // SBC >= 2:
                    swaitload(t // SBC)
                gather(t)
        swait(nch - 1)
        plsc.subcore_barrier()

        @pl.when(s < NS - 1)
        def _writeout_a():
            pltpu.sync_copy(acc_sh.at[pl.ds(base, RPT)],
                            out_hbm.at[c, pl.ds(base, RPT)])

        @pl.when(s == NS - 1)
        def _writeout_b():
            pltpu.sync_copy(acc_sh.at[pl.ds(base, RLAST)],
                            out_hbm.at[c, pl.ds(base, RLAST)])

    return k(vp, zeros, src1, dst4)


def _tc_prescale(degp, x):
    n, F = x.shape

    def body(degp_ref, x_ref, dinv_ref, xp_ref):
        deg = jnp.sum(degp_ref[...], axis=0, keepdims=True) + 1.0
        dinv = jnp.transpose(lax.rsqrt(deg))
        dinv_ref[...] = dinv
        xp_ref[...] = x_ref[...] * dinv

    return pl.pallas_call(
        body,
        out_shape=[jax.ShapeDtypeStruct((n, 1), jnp.float32),
                   jax.ShapeDtypeStruct((n, F), jnp.float32)],
    )(degp, x)


def _tc_mlp(acc, dinv, W1, b1, W2):
    _, n, F = acc.shape
    H = W1.shape[1]
    R = 2000

    def body(a0_ref, a1_ref, dinv_ref, W1_ref, b1_ref, W2_ref, gp_ref):
        agg = (a0_ref[0] + a1_ref[0]) * dinv_ref[...]
        h = jnp.dot(agg, W1_ref[...], preferred_element_type=jnp.float32)
        h = jnp.maximum(h + b1_ref[...], 0.0)
        g = jnp.dot(h, W2_ref[...], preferred_element_type=jnp.float32)
        gp_ref[...] = g * dinv_ref[...]

    return pl.pallas_call(
        body,
        grid=(n // R,),
        in_specs=[pl.BlockSpec((1, R, F), lambda i: (0, i, 0)),
                  pl.BlockSpec((1, R, F), lambda i: (1, i, 0)),
                  pl.BlockSpec((R, 1), lambda i: (i, 0)),
                  pl.BlockSpec((F, H), lambda i: (0, 0)),
                  pl.BlockSpec((1, H), lambda i: (0, 0)),
                  pl.BlockSpec((H, F), lambda i: (0, 0))],
        out_specs=pl.BlockSpec((R, F), lambda i: (i, 0)),
        out_shape=jax.ShapeDtypeStruct((n, F), jnp.float32),
    )(acc, acc, dinv, W1, b1, W2)


def _tc_final(acc, dinv, b2):
    _, n, F = acc.shape
    R = 2000

    def body(c0_ref, c1_ref, dinv_ref, b2_ref, out_ref):
        agg = (c0_ref[0] + c1_ref[0]) * dinv_ref[...]
        out_ref[...] = agg + b2_ref[...]

    return pl.pallas_call(
        body,
        grid=(n // R,),
        in_specs=[pl.BlockSpec((1, R, F), lambda i: (0, i, 0)),
                  pl.BlockSpec((1, R, F), lambda i: (1, i, 0)),
                  pl.BlockSpec((R, 1), lambda i: (i, 0)),
                  pl.BlockSpec((1, F), lambda i: (0, 0))],
        out_specs=pl.BlockSpec((R, F), lambda i: (i, 0)),
        out_shape=jax.ShapeDtypeStruct((n, F), jnp.float32),
    )(acc, acc, dinv, b2)


def kernel(x, edge_index, W1, b1, W2, b2):
    n, F = x.shape
    src1 = edge_index[0]
    dst4 = edge_index[1].reshape(NW, SB, SBC, CHUNK)
    zeros = jnp.zeros((n, F), jnp.float32)

    degp = _sc_degree(dst4, n)
    dinv, xp = _tc_prescale(degp, x)
    acc1 = _sc_aggregate(xp, zeros, src1, dst4)
    gp = _tc_mlp(acc1, dinv, W1, b1.reshape(1, -1), W2)
    acc2 = _sc_aggregate(gp, zeros, src1, dst4)
    out = _tc_final(acc2, dinv, b2.reshape(1, -1))
    return out
